# baseline (speedup 1.0000x reference)
import jax
import jax.numpy as jnp
from jax import lax
from jax.experimental import pallas as pl
from jax.experimental.pallas import tpu as pltpu
from jax.experimental.pallas import tpu_sc as plsc

_B = 1024
_L = 200
_D = 64

_NC = 2
_NS = 16
_NW = _NC * _NS

_BPW = _B // _NW
_GB = 2
_G = _BPW // _GB
_SPLITS = ((0, 104), (104, 96))


def _body(ids_hbm, table_hbm, pos_hbm, out_hbm,
          idx_nat, idx_bm, rows0, rows1, pos_v, gsem0, gsem1, ssem0, ssem1):
    c = lax.axis_index("c")
    s = lax.axis_index("s")
    wid = s * _NC + c
    j = wid // 4
    lane0 = (wid % 4) * _BPW

    rows = [rows0, rows1]
    gsem = [gsem0, gsem1]
    ssem = [ssem0, ssem1]

    pltpu.sync_copy(ids_hbm.at[:, j, :, pl.ds(lane0, _BPW)], idx_nat)
    pltpu.sync_copy(pos_hbm.at[pl.ds(0, _L)], pos_v)

    lane = jax.lax.iota(jnp.int32, 16)
    scatter_base = lane * _L

    def reorder_i(i, carry):
        for sub in range(8):
            l = i * 8 + sub
            for h in range(2):
                v = idx_nat[i, sub, pl.ds(h * 16, 16)]
                plsc.store_scatter(idx_bm, [scatter_base + (h * 16 * _L + l)], v)
        return carry

    lax.fori_loop(0, 25, reorder_i, 0)

    def issue_gathers(g):
        hs = []
        for b2 in range(_GB):
            lb = g * _GB + b2
            for off, size in _SPLITS:
                hs.append(pltpu.async_copy(
                    table_hbm.at[idx_bm.at[pl.ds(lb * _L + off, size)]],
                    rows[g % 2].at[b2, pl.ds(off, size)],
                    gsem[g % 2]))
        return hs

    gh = {0: issue_gathers(0)}
    sh = {}

    for g in range(_G):
        p = g % 2
        for h in gh[g]:
            h.wait()
        if g + 1 < _G:
            if g >= 1:
                sh[g - 1].wait()
            gh[g + 1] = issue_gathers(g + 1)

        rv = rows[p]

        def add_body(r, carry):
            for jj in range(_D // 16):
                pv = pos_v[r, pl.ds(jj * 16, 16)]
                for b2 in range(_GB):
                    plsc.addupdate(rv.at[b2, r, pl.ds(jj * 16, 16)], pv)
            return carry

        lax.fori_loop(0, _L, add_body, 0)

        sh[g] = pltpu.async_copy(
            rv, out_hbm.at[pl.ds(wid * _BPW + g * _GB, _GB)], ssem[p])

    sh[_G - 2].wait()
    sh[_G - 1].wait()


@jax.jit
def _embed(ids4, token_table, pos_table):
    mesh = plsc.VectorSubcoreMesh(core_axis_name="c", subcore_axis_name="s")
    f = pl.kernel(
        _body,
        out_type=jax.ShapeDtypeStruct((_B, _L, _D), jnp.float32),
        mesh=mesh,
        scratch_types=[
            pltpu.VMEM((_L // 8, 8, _BPW), jnp.int32),
            pltpu.VMEM((_BPW * _L,), jnp.int32),
            pltpu.VMEM((_GB, _L, _D), jnp.float32),
            pltpu.VMEM((_GB, _L, _D), jnp.float32),
            pltpu.VMEM((_L, _D), jnp.float32),
            pltpu.SemaphoreType.DMA,
            pltpu.SemaphoreType.DMA,
            pltpu.SemaphoreType.DMA,
            pltpu.SemaphoreType.DMA,
        ],
        compiler_params=pltpu.CompilerParams(
            use_tc_tiling_on_sc=False, needs_layout_passes=False),
    )
    return f(ids4, token_table, pos_table)


def kernel(input_ids, token_table, pos_table):
    ids4 = (input_ids.astype(jnp.int32).T
            .reshape(_L // 8, 8, _B // 128, 128)
            .transpose(0, 2, 1, 3))
    return _embed(ids4, token_table, pos_table)


# speedup vs baseline: 1.0012x; 1.0012x over previous
import jax
import jax.numpy as jnp
from jax import lax
from jax.experimental import pallas as pl
from jax.experimental.pallas import tpu as pltpu
from jax.experimental.pallas import tpu_sc as plsc

_B = 1024
_L = 200
_D = 64

_NC = 2
_NS = 16
_NW = _NC * _NS

_BPW = _B // _NW
_GB = 2
_G = _BPW // _GB
_SPLITS = ((0, 104), (104, 96))


def _body(ids_hbm, table_hbm, pos_hbm, out_hbm,
          idx_nat, idx_bm, rows0, rows1, pos_v, gsem0, gsem1, ssem0, ssem1):
    c = lax.axis_index("c")
    s = lax.axis_index("s")
    wid = s * _NC + c
    j = wid // 4
    lane0 = (wid % 4) * _BPW

    rows = [rows0, rows1]
    gsem = [gsem0, gsem1]
    ssem = [ssem0, ssem1]

    pltpu.sync_copy(ids_hbm.at[:, j, :, pl.ds(lane0, _BPW)], idx_nat)
    pltpu.sync_copy(pos_hbm.at[pl.ds(0, _L)], pos_v)

    lane = jax.lax.iota(jnp.int32, 16)
    scatter_base = lane * _L

    def reorder_i(i, carry):
        for sub in range(8):
            l = i * 8 + sub
            for h in range(2):
                v = idx_nat[i, sub, pl.ds(h * 16, 16)]
                plsc.store_scatter(idx_bm, [scatter_base + (h * 16 * _L + l)], v)
        return carry

    lax.fori_loop(0, 25, reorder_i, 0)

    def issue_gathers(g):
        hs = []
        for b2 in range(_GB):
            lb = g * _GB + b2
            for off, size in _SPLITS:
                hs.append(pltpu.async_copy(
                    table_hbm.at[idx_bm.at[pl.ds(lb * _L + off, size)]],
                    rows[g % 2].at[b2, pl.ds(off, size)],
                    gsem[g % 2]))
        return hs

    gh = {0: issue_gathers(0)}
    sh = {}

    for g in range(_G):
        p = g % 2
        for h in gh[g]:
            h.wait()
        if g + 1 < _G:
            if g >= 1:
                sh[g - 1].wait()
            gh[g + 1] = issue_gathers(g + 1)

        rv = rows[p]

        def add_body(r, carry):
            for jj in range(_D // 16):
                pv = pos_v[r, pl.ds(jj * 16, 16)]
                for b2 in range(_GB):
                    plsc.addupdate(rv.at[b2, r, pl.ds(jj * 16, 16)], pv)
            return carry

        lax.fori_loop(0, _L, add_body, 0)

        sh[g] = pltpu.async_copy(
            rv, out_hbm.at[pl.ds(wid * _BPW + g * _GB, _GB)], ssem[p])

    sh[_G - 2].wait()
    sh[_G - 1].wait()


@jax.jit
def _embed(ids4, token_table, pos_table):
    mesh = plsc.VectorSubcoreMesh(core_axis_name="c", subcore_axis_name="s")
    f = pl.kernel(
        _body,
        out_type=jax.ShapeDtypeStruct((_B, _L, _D), jnp.float32),
        mesh=mesh,
        scratch_types=[
            pltpu.VMEM((_L // 8, 8, _BPW), jnp.int32),
            pltpu.VMEM((_BPW * _L,), jnp.int32),
            pltpu.VMEM((_GB, _L, _D), jnp.float32),
            pltpu.VMEM((_GB, _L, _D), jnp.float32),
            pltpu.VMEM((_L, _D), jnp.float32),
            pltpu.SemaphoreType.DMA,
            pltpu.SemaphoreType.DMA,
            pltpu.SemaphoreType.DMA,
            pltpu.SemaphoreType.DMA,
        ],
        compiler_params=pltpu.CompilerParams(
            use_tc_tiling_on_sc=False, needs_layout_passes=False),
    )
    return f(ids4, token_table, pos_table)


def kernel(input_ids, token_table, pos_table):
    ids4 = (input_ids.astype(jnp.int32).T
            .reshape(_L // 8, 8, _B // 128, 128)
            .transpose(0, 2, 1, 3))
    ids4 = jnp.clip(ids4, 0, 999999)
    return _embed(ids4, token_table, pos_table)
